# Initial kernel scaffold; baseline (speedup 1.0000x reference)
import jax
import jax.numpy as jnp
from jax.experimental import pallas as pl


def kernel(indices, embeddings):
    raise NotImplementedError("write your pallas kernel here")


# speedup vs baseline: 1.6867x; 1.6867x over previous
import functools

import jax
import jax.numpy as jnp
from jax import lax
from jax.experimental import pallas as pl
from jax.experimental.pallas import tpu as pltpu
from jax.experimental.pallas import tpu_sc as plsc

D = 64
TOTAL = 16384 * 50
NC, NS = 2, 16
NW = NC * NS
PER_W = TOTAL // NW
CH = 128
NCHUNK = PER_W // CH

_mesh = plsc.VectorSubcoreMesh(core_axis_name="c", subcore_axis_name="s")


@functools.partial(
    pl.kernel,
    mesh=_mesh,
    out_type=jax.ShapeDtypeStruct((TOTAL, D), jnp.float32),
    compiler_params=pltpu.CompilerParams(use_tc_tiling_on_sc=False),
    scratch_types=[
        pltpu.VMEM((NCHUNK, CH), jnp.int32),
        pltpu.VMEM((CH, D), jnp.float32),
        pltpu.SemaphoreType.DMA,
    ],
)
def _gather_kernel(idx_hbm, table_hbm, out_hbm, idx_v, rows_v, gsem):
    wid = lax.axis_index("s") * NC + lax.axis_index("c")
    base = wid * PER_W
    pltpu.sync_copy(idx_hbm.at[wid], idx_v)

    def body(c, carry):
        pltpu.async_copy(table_hbm.at[idx_v.at[c]], rows_v, gsem).wait()
        pltpu.sync_copy(rows_v, out_hbm.at[pl.ds(base + c * CH, CH)])
        return carry

    lax.fori_loop(0, NCHUNK, body, 0)


def kernel(indices, embeddings):
    idx = indices.reshape(NW, NCHUNK, CH).astype(jnp.int32)
    out = _gather_kernel(idx, embeddings)
    return out.reshape(*indices.shape, D)


# speedup vs baseline: 1.8661x; 1.1064x over previous
import functools

import jax
import jax.numpy as jnp
from jax import lax
from jax.experimental import pallas as pl
from jax.experimental.pallas import tpu as pltpu
from jax.experimental.pallas import tpu_sc as plsc

D = 64
TOTAL = 16384 * 50
NC, NS = 2, 16
NW = NC * NS
PER_W = TOTAL // NW
CH = 128
NCHUNK = PER_W // CH
NBUF = 4
K = 2
NG = NCHUNK // NBUF

_mesh = plsc.VectorSubcoreMesh(core_axis_name="c", subcore_axis_name="s")


@functools.partial(
    pl.kernel,
    mesh=_mesh,
    out_type=jax.ShapeDtypeStruct((TOTAL, D), jnp.float32),
    compiler_params=pltpu.CompilerParams(use_tc_tiling_on_sc=False),
    scratch_types=[
        pltpu.VMEM((NCHUNK, CH), jnp.int32),
        [pltpu.VMEM((CH, D), jnp.float32) for _ in range(NBUF)],
        [pltpu.SemaphoreType.DMA for _ in range(NBUF)],
        [pltpu.SemaphoreType.DMA for _ in range(NBUF)],
    ],
)
def _gather_kernel(idx_hbm, table_hbm, out_hbm, idx_v, rows, gsem, ssem):
    wid = lax.axis_index("s") * NC + lax.axis_index("c")
    base = wid * PER_W
    pltpu.sync_copy(idx_hbm.at[wid], idx_v)

    def fire_g(b, c):
        pltpu.async_copy(table_hbm.at[idx_v.at[c]], rows[b], gsem[b])

    def wait_g(b):
        pltpu.make_async_copy(table_hbm.at[idx_v.at[0]], rows[b], gsem[b]).wait()

    def fire_s(b, c):
        pltpu.async_copy(rows[b], out_hbm.at[pl.ds(base + c * CH, CH)], ssem[b])

    def wait_s(b):
        pltpu.make_async_copy(rows[b], out_hbm.at[pl.ds(base, CH)], ssem[b]).wait()

    fire_g(0, 0)
    fire_g(1, 1)
    wait_g(0); fire_s(0, 0); fire_g(2, 2)
    wait_g(1); fire_s(1, 1); fire_g(3, 3)
    wait_g(2); fire_s(2, 2); wait_s(0); fire_g(0, 4)
    wait_g(3); fire_s(3, 3); wait_s(1); fire_g(1, 5)

    def group(g, carry):
        for b in range(NBUF):
            c = g * NBUF + b
            wait_g(b)
            fire_s(b, c)
            bf = (b + K) % NBUF
            wait_s(bf)
            fire_g(bf, c + K)
        return carry

    lax.fori_loop(1, NG - 1, group, 0)

    c0 = (NG - 1) * NBUF
    wait_g(0); fire_s(0, c0 + 0); wait_s(2); fire_g(2, c0 + 2)
    wait_g(1); fire_s(1, c0 + 1); wait_s(3); fire_g(3, c0 + 3)
    wait_g(2); fire_s(2, c0 + 2)
    wait_g(3); fire_s(3, c0 + 3)
    for b in range(NBUF):
        wait_s(b)


def kernel(indices, embeddings):
    idx = indices.reshape(NW, NCHUNK, CH).astype(jnp.int32)
    out = _gather_kernel(idx, embeddings)
    return out.reshape(*indices.shape, D)


# speedup vs baseline: 1.8755x; 1.0050x over previous
import functools

import jax
import jax.numpy as jnp
from jax import lax
from jax.experimental import pallas as pl
from jax.experimental.pallas import tpu as pltpu
from jax.experimental.pallas import tpu_sc as plsc

D = 64
TOTAL = 16384 * 50
NC, NS = 2, 16
NW = NC * NS
PER_W = TOTAL // NW
CH = 256
NCHUNK = PER_W // CH
NBUF = 4
K = 2
NG = NCHUNK // NBUF

_mesh = plsc.VectorSubcoreMesh(core_axis_name="c", subcore_axis_name="s")


@functools.partial(
    pl.kernel,
    mesh=_mesh,
    out_type=jax.ShapeDtypeStruct((TOTAL, D), jnp.float32),
    compiler_params=pltpu.CompilerParams(use_tc_tiling_on_sc=False),
    scratch_types=[
        pltpu.VMEM((NCHUNK, CH), jnp.int32),
        [pltpu.VMEM((CH, D), jnp.float32) for _ in range(NBUF)],
        [pltpu.SemaphoreType.DMA for _ in range(NBUF)],
        [pltpu.SemaphoreType.DMA for _ in range(NBUF)],
    ],
)
def _gather_kernel(idx_hbm, table_hbm, out_hbm, idx_v, rows, gsem, ssem):
    wid = lax.axis_index("s") * NC + lax.axis_index("c")
    base = wid * PER_W
    pltpu.sync_copy(idx_hbm.at[wid], idx_v)

    def fire_g(b, c):
        pltpu.async_copy(table_hbm.at[idx_v.at[c]], rows[b], gsem[b])

    def wait_g(b):
        pltpu.make_async_copy(table_hbm.at[idx_v.at[0]], rows[b], gsem[b]).wait()

    def fire_s(b, c):
        pltpu.async_copy(rows[b], out_hbm.at[pl.ds(base + c * CH, CH)], ssem[b])

    def wait_s(b):
        pltpu.make_async_copy(rows[b], out_hbm.at[pl.ds(base, CH)], ssem[b]).wait()

    fire_g(0, 0)
    fire_g(1, 1)
    wait_g(0); fire_s(0, 0); fire_g(2, 2)
    wait_g(1); fire_s(1, 1); fire_g(3, 3)
    wait_g(2); fire_s(2, 2); wait_s(0); fire_g(0, 4)
    wait_g(3); fire_s(3, 3); wait_s(1); fire_g(1, 5)

    def group(g, carry):
        for b in range(NBUF):
            c = g * NBUF + b
            wait_g(b)
            fire_s(b, c)
            bf = (b + K) % NBUF
            wait_s(bf)
            fire_g(bf, c + K)
        return carry

    lax.fori_loop(1, NG - 1, group, 0)

    c0 = (NG - 1) * NBUF
    wait_g(0); fire_s(0, c0 + 0); wait_s(2); fire_g(2, c0 + 2)
    wait_g(1); fire_s(1, c0 + 1); wait_s(3); fire_g(3, c0 + 3)
    wait_g(2); fire_s(2, c0 + 2)
    wait_g(3); fire_s(3, c0 + 3)
    for b in range(NBUF):
        wait_s(b)


def kernel(indices, embeddings):
    idx = indices.reshape(NW, NCHUNK, CH).astype(jnp.int32)
    out = _gather_kernel(idx, embeddings)
    return out.reshape(*indices.shape, D)
